# baseline (speedup 1.0000x reference)
import jax
import jax.numpy as jnp
from jax import lax
from jax.experimental import pallas as pl
from jax.experimental.pallas import tpu as pltpu
from jax.experimental.pallas import tpu_sc as plsc

_C0 = 10000
_C1 = 100
_D = 32
_BATCH = 16384

_NC = 2
_NS = 16
_NW = _NC * _NS
_BPW = _BATCH // _NW
_CHUNK = 128
_NCHUNK = _BPW // _CHUNK
_L = 16


def _sc_body(uids_hbm, iids_hbm,
             user_obj, user_c0, user_c1,
             item_obj, item_c0, item_c1,
             out_hbm,
             uid_v, iid_v, uc0_v, uc1_v, ic0_v, ic1_v,
             uo_r, uc0_r, uc1_r, io_r, ic0_r, ic1_r,
             out_v, sem):
    wid = lax.axis_index("s") * _NC + lax.axis_index("c")
    base = wid * _BPW

    pltpu.sync_copy(uids_hbm.at[pl.ds(base, _BPW)], uid_v)
    pltpu.sync_copy(iids_hbm.at[pl.ds(base, _BPW)], iid_v)

    def _idx_body(g, _):
        sl = pl.ds(g * _L, _L)
        u = uid_v[sl]
        i = iid_v[sl]
        uc0_v[sl] = lax.rem(u, _C0)
        uc1_v[sl] = lax.rem(u, _C1)
        ic0_v[sl] = lax.rem(i, _C0)
        ic1_v[sl] = lax.rem(i, _C1)
        return 0

    lax.fori_loop(0, _BPW // _L, _idx_body, 0)

    copies = []
    for tab, idx, dst in (
        (user_obj, uid_v, uo_r),
        (user_c0, uc0_v, uc0_r),
        (user_c1, uc1_v, uc1_r),
        (item_obj, iid_v, io_r),
        (item_c0, ic0_v, ic0_r),
        (item_c1, ic1_v, ic1_r),
    ):
        for c in range(_NCHUNK):
            sl = pl.ds(c * _CHUNK, _CHUNK)
            copies.append(
                pltpu.make_async_copy(tab.at[idx.at[sl]], dst.at[sl], sem))
    for cp in copies:
        cp.start()
    for cp in copies:
        cp.wait()

    lanes = lax.iota(jnp.int32, _L)

    def _dot_body(g, _):
        acc = jnp.zeros((_L,), jnp.float32)
        for r16 in range(_L):
            r = g * _L + r16
            ulo = jnp.zeros((_L,), jnp.float32)
            uhi = jnp.zeros((_L,), jnp.float32)
            for ref in (uo_r, uc0_r, uc1_r):
                lo, hi = plsc.unpack(ref[r, :], format=plsc.PackFormat.INTERLEAVED)
                ulo = ulo + lo
                uhi = uhi + hi
            vlo = jnp.zeros((_L,), jnp.float32)
            vhi = jnp.zeros((_L,), jnp.float32)
            for ref in (io_r, ic0_r, ic1_r):
                lo, hi = plsc.unpack(ref[r, :], format=plsc.PackFormat.INTERLEAVED)
                vlo = vlo + lo
                vhi = vhi + hi
            p = ulo * vlo + uhi * vhi
            acc = jnp.where(lanes == r16, jnp.sum(p), acc)
        out_v[pl.ds(g * _L, _L)] = acc
        return 0

    lax.fori_loop(0, _BPW // _L, _dot_body, 0)

    pltpu.sync_copy(out_v, out_hbm.at[pl.ds(base, _BPW)])


def kernel(X, user_obj, user_c0, user_c1, item_obj, item_c0, item_c1):
    uids = X[:, 0]
    iids = X[:, 1]

    bf = jnp.bfloat16
    k = pl.kernel(
        _sc_body,
        out_type=jax.ShapeDtypeStruct((_BATCH,), jnp.float32),
        mesh=plsc.VectorSubcoreMesh(core_axis_name="c", subcore_axis_name="s"),
        compiler_params=pltpu.CompilerParams(
            needs_layout_passes=False, use_tc_tiling_on_sc=False),
        scratch_types=[
            pltpu.VMEM((_BPW,), jnp.int32),
            pltpu.VMEM((_BPW,), jnp.int32),
            pltpu.VMEM((_BPW,), jnp.int32),
            pltpu.VMEM((_BPW,), jnp.int32),
            pltpu.VMEM((_BPW,), jnp.int32),
            pltpu.VMEM((_BPW,), jnp.int32),
            pltpu.VMEM((_BPW, _D), bf),
            pltpu.VMEM((_BPW, _D), bf),
            pltpu.VMEM((_BPW, _D), bf),
            pltpu.VMEM((_BPW, _D), bf),
            pltpu.VMEM((_BPW, _D), bf),
            pltpu.VMEM((_BPW, _D), bf),
            pltpu.VMEM((_BPW,), jnp.float32),
            pltpu.SemaphoreType.DMA,
        ],
    )
    out = k(uids, iids,
            user_obj.astype(bf), user_c0.astype(bf), user_c1.astype(bf),
            item_obj.astype(bf), item_c0.astype(bf), item_c1.astype(bf))
    return out.reshape(_BATCH, 1)


# speedup vs baseline: 3.7820x; 3.7820x over previous
import jax
import jax.numpy as jnp
from jax import lax
from jax.experimental import pallas as pl
from jax.experimental.pallas import tpu as pltpu
from jax.experimental.pallas import tpu_sc as plsc

_C0 = 10000
_C1 = 100
_D = 32
_BATCH = 16384
_PK = 128 // _D

_NC = 2
_NS = 16
_NW = _NC * _NS
_BPW = _BATCH // _NW
_CHUNK = 128
_NCHUNK = _BPW // _CHUNK
_L = 16
_W = 8


def _sc_body(uids_hbm, iids_hbm,
             uo_t, io_t, uc0_p, ic0_p, uc1_p, ic1_p,
             out_hbm,
             uid_v, iid_v, uc0q_v, ic0q_v,
             uw_b, iw_b, uc0_b, ic0_b, uc1_v, ic1_v,
             out_v, sem, wsem):
    wid = lax.axis_index("s") * _NC + lax.axis_index("c")
    base = wid * _BPW

    pltpu.sync_copy(uids_hbm.at[pl.ds(base, _BPW)], uid_v)
    pltpu.sync_copy(iids_hbm.at[pl.ds(base, _BPW)], iid_v)
    pltpu.sync_copy(uc1_p, uc1_v)
    pltpu.sync_copy(ic1_p, ic1_v)

    def _idx_body(g, _):
        sl = pl.ds(g * _L, _L)
        uc0q_v[sl] = lax.shift_right_logical(lax.rem(uid_v[sl], _C0), 2)
        ic0q_v[sl] = lax.shift_right_logical(lax.rem(iid_v[sl], _C0), 2)
        return 0

    lax.fori_loop(0, _BPW // _L, _idx_body, 0)

    lanes = lax.iota(jnp.int32, _L)
    rows16 = lax.iota(jnp.int32, _L)

    for c in range(_NCHUNK):
        csl = pl.ds(c * _CHUNK, _CHUNK)
        c0_copies = [
            pltpu.make_async_copy(uc0_p.at[uc0q_v.at[csl]], uc0_b, sem),
            pltpu.make_async_copy(ic0_p.at[ic0q_v.at[csl]], ic0_b, sem),
        ]
        for cp in c0_copies:
            cp.start()
        for cp in c0_copies:
            cp.wait()

        def _grp_body(g, _):
            gsl = pl.ds(c * _CHUNK + g * _L, _L)
            uvec = uid_v[gsl]
            ivec = iid_v[gsl]
            ublk = lax.shift_right_logical(uvec, 7)
            iblk = lax.shift_right_logical(ivec, 7)
            ucol = uvec & 127
            icol = ivec & 127
            uoff = (lax.rem(uvec, _C0) & (_PK - 1)) * _D
            ioff = (lax.rem(ivec, _C0) & (_PK - 1)) * _D
            uq1 = lax.shift_right_logical(lax.rem(uvec, _C1), 2)
            iq1 = lax.shift_right_logical(lax.rem(ivec, _C1), 2)
            uo1 = (lax.rem(uvec, _C1) & (_PK - 1)) * _D
            io1 = (lax.rem(ivec, _C1) & (_PK - 1)) * _D

            acc = jnp.zeros((_L,), jnp.float32)
            for wv in range(_L // _W):
                wcopies = []
                for k in range(_W):
                    r16 = wv * _W + k
                    ub = pl.multiple_of(ublk[r16] * 128, 128)
                    ib = pl.multiple_of(iblk[r16] * 128, 128)
                    wcopies.append(pltpu.make_async_copy(
                        uo_t.at[:, pl.ds(ub, 128)], uw_b.at[k], wsem))
                    wcopies.append(pltpu.make_async_copy(
                        io_t.at[:, pl.ds(ib, 128)], iw_b.at[k], wsem))
                for cp in wcopies:
                    cp.start()
                for cp in wcopies:
                    cp.wait()

                for k in range(_W):
                    r16 = wv * _W + k
                    rloc = g * _L + r16
                    kv = jnp.full((_L,), k, jnp.int32)
                    ucols = jnp.full((_L,), ucol[r16], jnp.int32)
                    icols = jnp.full((_L,), icol[r16], jnp.int32)
                    p = jnp.zeros((_L,), jnp.float32)
                    for h in range(_D // _L):
                        hrows = rows16 + h * _L
                        us = pl.ds(uoff[r16] + h * _L, _L)
                        vs = pl.ds(ioff[r16] + h * _L, _L)
                        u1s = pl.ds(uo1[r16] + h * _L, _L)
                        v1s = pl.ds(io1[r16] + h * _L, _L)
                        u = (plsc.load_gather(uw_b, [kv, hrows, ucols])
                             + uc0_b[rloc, us]
                             + uc1_v[uq1[r16], u1s])
                        v = (plsc.load_gather(iw_b, [kv, hrows, icols])
                             + ic0_b[rloc, vs]
                             + ic1_v[iq1[r16], v1s])
                        p = p + u * v
                    acc = jnp.where(lanes == r16, jnp.sum(p), acc)
            out_v[gsl] = acc
            return 0

        lax.fori_loop(0, _CHUNK // _L, _grp_body, 0)

    pltpu.sync_copy(out_v, out_hbm.at[pl.ds(base, _BPW)])


def kernel(X, user_obj, user_c0, user_c1, item_obj, item_c0, item_c1):
    uids = X[:, 0]
    iids = X[:, 1]

    uo_t = user_obj.T
    io_t = item_obj.T
    uc0_p = user_c0.reshape(-1, 128)
    ic0_p = item_c0.reshape(-1, 128)
    uc1_p = user_c1.reshape(-1, 128)
    ic1_p = item_c1.reshape(-1, 128)

    k = pl.kernel(
        _sc_body,
        out_type=jax.ShapeDtypeStruct((_BATCH,), jnp.float32),
        mesh=plsc.VectorSubcoreMesh(core_axis_name="c", subcore_axis_name="s"),
        compiler_params=pltpu.CompilerParams(needs_layout_passes=False),
        scratch_types=[
            pltpu.VMEM((_BPW,), jnp.int32),
            pltpu.VMEM((_BPW,), jnp.int32),
            pltpu.VMEM((_BPW,), jnp.int32),
            pltpu.VMEM((_BPW,), jnp.int32),
            pltpu.VMEM((_W, _D, 128), jnp.float32),
            pltpu.VMEM((_W, _D, 128), jnp.float32),
            pltpu.VMEM((_CHUNK, 128), jnp.float32),
            pltpu.VMEM((_CHUNK, 128), jnp.float32),
            pltpu.VMEM((_C1 // _PK, 128), jnp.float32),
            pltpu.VMEM((_C1 // _PK, 128), jnp.float32),
            pltpu.VMEM((_BPW,), jnp.float32),
            pltpu.SemaphoreType.DMA,
            pltpu.SemaphoreType.DMA,
        ],
    )
    out = k(uids, iids, uo_t, io_t, uc0_p, ic0_p, uc1_p, ic1_p)
    return out.reshape(_BATCH, 1)


# speedup vs baseline: 3.7932x; 1.0030x over previous
import jax
import jax.numpy as jnp
from jax import lax
from jax.experimental import pallas as pl
from jax.experimental.pallas import tpu as pltpu
from jax.experimental.pallas import tpu_sc as plsc

_C0 = 10000
_C1 = 100
_D = 32
_BATCH = 16384
_PK = 128 // _D

_NC = 2
_NS = 16
_NW = _NC * _NS
_BPW = _BATCH // _NW
_CHUNK = 128
_NCHUNK = _BPW // _CHUNK
_L = 16
_W = 8


def _sc_body(uids_hbm, iids_hbm,
             uo_t, io_t, uc0_p, ic0_p, uc1_p, ic1_p,
             out_hbm,
             uid_v, iid_v, uc0q_v, ic0q_v,
             uw_b, iw_b, uc0_b, ic0_b, uc1_v, ic1_v,
             out_v, sem, wsem):
    wid = lax.axis_index("s") * _NC + lax.axis_index("c")
    base = wid * _BPW

    pltpu.sync_copy(uids_hbm.at[pl.ds(base, _BPW)], uid_v)
    pltpu.sync_copy(iids_hbm.at[pl.ds(base, _BPW)], iid_v)
    pltpu.sync_copy(uc1_p, uc1_v)
    pltpu.sync_copy(ic1_p, ic1_v)

    def _idx_body(g, _):
        sl = pl.ds(g * _L, _L)
        uc0q_v[sl] = lax.shift_right_logical(lax.rem(uid_v[sl], _C0), 2)
        ic0q_v[sl] = lax.shift_right_logical(lax.rem(iid_v[sl], _C0), 2)
        return 0

    lax.fori_loop(0, _BPW // _L, _idx_body, 0)

    lanes = lax.iota(jnp.int32, _L)
    rows16 = lax.iota(jnp.int32, _L)

    for c in range(_NCHUNK):
        csl = pl.ds(c * _CHUNK, _CHUNK)
        c0_copies = [
            pltpu.make_async_copy(uc0_p.at[uc0q_v.at[csl]], uc0_b, sem),
            pltpu.make_async_copy(ic0_p.at[ic0q_v.at[csl]], ic0_b, sem),
        ]
        for cp in c0_copies:
            cp.start()
        for cp in c0_copies:
            cp.wait()

        def _grp_body(g, _):
            gsl = pl.ds(c * _CHUNK + g * _L, _L)
            uvec = uid_v[gsl]
            ivec = iid_v[gsl]
            ublk = lax.shift_right_logical(uvec, 7)
            iblk = lax.shift_right_logical(ivec, 7)
            ucol = uvec & 127
            icol = ivec & 127
            uoff = (lax.rem(uvec, _C0) & (_PK - 1)) * _D
            ioff = (lax.rem(ivec, _C0) & (_PK - 1)) * _D
            uq1 = lax.shift_right_logical(lax.rem(uvec, _C1), 2)
            iq1 = lax.shift_right_logical(lax.rem(ivec, _C1), 2)
            uo1 = (lax.rem(uvec, _C1) & (_PK - 1)) * _D
            io1 = (lax.rem(ivec, _C1) & (_PK - 1)) * _D

            acc = jnp.zeros((_L,), jnp.float32)
            for wv in range(_L // _W):
                wcopies = []
                for k in range(_W):
                    r16 = wv * _W + k
                    ub = pl.multiple_of(ublk[r16] * 128, 128)
                    ib = pl.multiple_of(iblk[r16] * 128, 128)
                    for tj in range(_D // 8):
                        tsl = pl.ds(tj * 8, 8)
                        wcopies.append(pltpu.make_async_copy(
                            uo_t.at[tsl, pl.ds(ub, 128)],
                            uw_b.at[k, tsl], wsem))
                        wcopies.append(pltpu.make_async_copy(
                            io_t.at[tsl, pl.ds(ib, 128)],
                            iw_b.at[k, tsl], wsem))
                for cp in wcopies:
                    cp.start()
                for cp in wcopies:
                    cp.wait()

                for k in range(_W):
                    r16 = wv * _W + k
                    rloc = g * _L + r16
                    kv = jnp.full((_L,), k, jnp.int32)
                    ucols = jnp.full((_L,), ucol[r16], jnp.int32)
                    icols = jnp.full((_L,), icol[r16], jnp.int32)
                    p = jnp.zeros((_L,), jnp.float32)
                    for h in range(_D // _L):
                        hrows = rows16 + h * _L
                        us = pl.ds(uoff[r16] + h * _L, _L)
                        vs = pl.ds(ioff[r16] + h * _L, _L)
                        u1s = pl.ds(uo1[r16] + h * _L, _L)
                        v1s = pl.ds(io1[r16] + h * _L, _L)
                        u = (plsc.load_gather(uw_b, [kv, hrows, ucols])
                             + uc0_b[rloc, us]
                             + uc1_v[uq1[r16], u1s])
                        v = (plsc.load_gather(iw_b, [kv, hrows, icols])
                             + ic0_b[rloc, vs]
                             + ic1_v[iq1[r16], v1s])
                        p = p + u * v
                    acc = jnp.where(lanes == r16, jnp.sum(p), acc)
            out_v[gsl] = acc
            return 0

        lax.fori_loop(0, _CHUNK // _L, _grp_body, 0)

    pltpu.sync_copy(out_v, out_hbm.at[pl.ds(base, _BPW)])


def kernel(X, user_obj, user_c0, user_c1, item_obj, item_c0, item_c1):
    uids = X[:, 0]
    iids = X[:, 1]

    uo_t = user_obj.T
    io_t = item_obj.T
    uc0_p = user_c0.reshape(-1, 128)
    ic0_p = item_c0.reshape(-1, 128)
    uc1_p = user_c1.reshape(-1, 128)
    ic1_p = item_c1.reshape(-1, 128)

    k = pl.kernel(
        _sc_body,
        out_type=jax.ShapeDtypeStruct((_BATCH,), jnp.float32),
        mesh=plsc.VectorSubcoreMesh(core_axis_name="c", subcore_axis_name="s"),
        compiler_params=pltpu.CompilerParams(needs_layout_passes=False),
        scratch_types=[
            pltpu.VMEM((_BPW,), jnp.int32),
            pltpu.VMEM((_BPW,), jnp.int32),
            pltpu.VMEM((_BPW,), jnp.int32),
            pltpu.VMEM((_BPW,), jnp.int32),
            pltpu.VMEM((_W, _D, 128), jnp.float32),
            pltpu.VMEM((_W, _D, 128), jnp.float32),
            pltpu.VMEM((_CHUNK, 128), jnp.float32),
            pltpu.VMEM((_CHUNK, 128), jnp.float32),
            pltpu.VMEM((_C1 // _PK, 128), jnp.float32),
            pltpu.VMEM((_C1 // _PK, 128), jnp.float32),
            pltpu.VMEM((_BPW,), jnp.float32),
            pltpu.SemaphoreType.DMA,
            pltpu.SemaphoreType.DMA,
        ],
    )
    out = k(uids, iids, uo_t, io_t, uc0_p, ic0_p, uc1_p, ic1_p)
    return out.reshape(_BATCH, 1)
